# baseline (speedup 1.0000x reference)
import functools

import jax
import jax.numpy as jnp
from jax import lax
from jax.experimental import pallas as pl
from jax.experimental.pallas import tpu as pltpu
from jax.experimental.pallas import tpu_sc as plsc

B = 16384
D = 32
PACK = 4
LINES = 1000000 // PACK
NC = 2
NS = 16
NW = NC * NS
BPW = B // NW
HALF = BPW // 2


def _sc_gather_lines(uidx, iidx, utab_l, itab_l):
    mesh = plsc.VectorSubcoreMesh(core_axis_name="c", subcore_axis_name="s")

    @functools.partial(
        pl.kernel,
        mesh=mesh,
        out_type=(
            jax.ShapeDtypeStruct((B, 128), jnp.float32),
            jax.ShapeDtypeStruct((B, 128), jnp.float32),
        ),
        scratch_types=[
            pltpu.VMEM((BPW,), jnp.int32),
            pltpu.VMEM((BPW,), jnp.int32),
            pltpu.VMEM((HALF, 128), jnp.float32),
            pltpu.VMEM((HALF, 128), jnp.float32),
            pltpu.SemaphoreType.DMA,
            pltpu.SemaphoreType.DMA,
        ],
    )
    def k(uidx_hbm, iidx_hbm, utab_hbm, itab_hbm, ou_hbm, oi_hbm,
          uidx_v, iidx_v, buf_a, buf_b, sem_a, sem_b):
        wid = lax.axis_index("s") * NC + lax.axis_index("c")
        base = wid * BPW
        pltpu.sync_copy(uidx_hbm.at[pl.ds(base, BPW)], uidx_v)
        pltpu.sync_copy(iidx_hbm.at[pl.ds(base, BPW)], iidx_v)
        ca = pltpu.async_copy(utab_hbm.at[uidx_v.at[pl.ds(0, HALF)]], buf_a, sem_a)
        cb = pltpu.async_copy(utab_hbm.at[uidx_v.at[pl.ds(HALF, HALF)]], buf_b, sem_b)
        ca.wait()
        pltpu.sync_copy(buf_a, ou_hbm.at[pl.ds(base, HALF)])
        ca = pltpu.async_copy(itab_hbm.at[iidx_v.at[pl.ds(0, HALF)]], buf_a, sem_a)
        cb.wait()
        pltpu.sync_copy(buf_b, ou_hbm.at[pl.ds(base + HALF, HALF)])
        cb = pltpu.async_copy(itab_hbm.at[iidx_v.at[pl.ds(HALF, HALF)]], buf_b, sem_b)
        ca.wait()
        pltpu.sync_copy(buf_a, oi_hbm.at[pl.ds(base, HALF)])
        cb.wait()
        pltpu.sync_copy(buf_b, oi_hbm.at[pl.ds(base + HALF, HALF)])

    return k(uidx, iidx, utab_l, itab_l)


def _tc_head_body(lu_ref, li_ref, wu_ref, wi_ref, uq_ref, iq_ref, b_ref, o_ref):
    pu = jnp.dot(lu_ref[...], wu_ref[...], preferred_element_type=jnp.float32)
    pi = jnp.dot(li_ref[...], wi_ref[...], preferred_element_type=jnp.float32)
    quad = lax.broadcasted_iota(jnp.int32, (1, PACK), 1)
    su = jnp.sum(jnp.where(uq_ref[...] == quad, pu, 0.0), axis=1, keepdims=True)
    si = jnp.sum(jnp.where(iq_ref[...] == quad, pi, 0.0), axis=1, keepdims=True)
    o_ref[...] = jax.nn.sigmoid(su + si + b_ref[0])


def _tc_head(lu, li, w4u, w4i, uq, iq, fc_b):
    blk = 2048
    return pl.pallas_call(
        _tc_head_body,
        grid=(B // blk,),
        in_specs=[
            pl.BlockSpec((blk, 128), lambda i: (i, 0)),
            pl.BlockSpec((blk, 128), lambda i: (i, 0)),
            pl.BlockSpec((128, PACK), lambda i: (0, 0)),
            pl.BlockSpec((128, PACK), lambda i: (0, 0)),
            pl.BlockSpec((blk, 1), lambda i: (i, 0)),
            pl.BlockSpec((blk, 1), lambda i: (i, 0)),
            pl.BlockSpec(memory_space=pltpu.SMEM),
        ],
        out_specs=pl.BlockSpec((blk, 1), lambda i: (i, 0)),
        out_shape=jax.ShapeDtypeStruct((B, 1), jnp.float32),
    )(lu, li, w4u, w4i, uq, iq, fc_b)


def kernel(user_ids, item_ids, user_table, item_table, fc_w, fc_b):
    utab_l = user_table.reshape(LINES, PACK * D)
    itab_l = item_table.reshape(LINES, PACK * D)
    uidx = (user_ids // PACK).astype(jnp.int32)
    iidx = (item_ids // PACK).astype(jnp.int32)
    uq = (user_ids % PACK).astype(jnp.int32).reshape(B, 1)
    iq = (item_ids % PACK).astype(jnp.int32).reshape(B, 1)
    wu = fc_w[0, :D]
    wi = fc_w[0, D:]
    eye = jnp.eye(PACK, dtype=jnp.float32)
    w4u = jnp.kron(eye, wu.reshape(D, 1))
    w4i = jnp.kron(eye, wi.reshape(D, 1))
    lu, li = _sc_gather_lines(uidx, iidx, utab_l, itab_l)
    return _tc_head(lu, li, w4u, w4i, uq, iq, fc_b)


# speedup vs baseline: 8.8138x; 8.8138x over previous
import functools

import jax
import jax.numpy as jnp
from jax import lax
from jax.experimental import pallas as pl
from jax.experimental.pallas import tpu as pltpu
from jax.experimental.pallas import tpu_sc as plsc

B = 16384
D = 32
V = 1000000
NC = 2
NS = 16
NW = NC * NS
BPW = B // NW
VBLK = 65536
VL = 16


def _tc_scores_body(ut_ref, it_ref, wu_ref, wi_ref, b_ref, su_ref, si_ref):
    su_ref[...] = jnp.sum(ut_ref[...] * wu_ref[...], axis=0) + b_ref[0]
    si_ref[...] = jnp.sum(it_ref[...] * wi_ref[...], axis=0)


def _tc_scores(ut_t, it_t, fc_w, fc_b):
    wu = fc_w[0, :D].reshape(D, 1)
    wi = fc_w[0, D:].reshape(D, 1)
    grid = (V + VBLK - 1) // VBLK
    return pl.pallas_call(
        _tc_scores_body,
        grid=(grid,),
        in_specs=[
            pl.BlockSpec((D, VBLK), lambda i: (0, i)),
            pl.BlockSpec((D, VBLK), lambda i: (0, i)),
            pl.BlockSpec((D, 1), lambda i: (0, 0)),
            pl.BlockSpec((D, 1), lambda i: (0, 0)),
            pl.BlockSpec(memory_space=pltpu.SMEM),
        ],
        out_specs=(
            pl.BlockSpec((VBLK,), lambda i: (i,)),
            pl.BlockSpec((VBLK,), lambda i: (i,)),
        ),
        out_shape=(
            jax.ShapeDtypeStruct((V,), jnp.float32),
            jax.ShapeDtypeStruct((V,), jnp.float32),
        ),
    )(ut_t, it_t, wu, wi, fc_b)


def _sc_gather_head(user_ids, item_ids, score_u, score_i):
    mesh = plsc.VectorSubcoreMesh(core_axis_name="c", subcore_axis_name="s")

    @functools.partial(
        pl.kernel,
        mesh=mesh,
        out_type=jax.ShapeDtypeStruct((B,), jnp.float32),
        scratch_types=[
            pltpu.VMEM((BPW,), jnp.int32),
            pltpu.VMEM((BPW,), jnp.int32),
            pltpu.VMEM((BPW,), jnp.float32),
            pltpu.VMEM((BPW,), jnp.float32),
            pltpu.VMEM((BPW,), jnp.float32),
            pltpu.SemaphoreType.DMA,
            pltpu.SemaphoreType.DMA,
        ],
    )
    def k(uid_hbm, iid_hbm, su_hbm, si_hbm, o_hbm,
          uidx_v, iidx_v, su_v, si_v, o_v, sem_u, sem_i):
        wid = lax.axis_index("s") * NC + lax.axis_index("c")
        base = wid * BPW
        pltpu.sync_copy(uid_hbm.at[pl.ds(base, BPW)], uidx_v)
        pltpu.sync_copy(iid_hbm.at[pl.ds(base, BPW)], iidx_v)
        cu = pltpu.async_copy(su_hbm.at[uidx_v], su_v, sem_u)
        ci = pltpu.async_copy(si_hbm.at[iidx_v], si_v, sem_i)
        cu.wait()
        ci.wait()

        @pl.loop(0, BPW, step=VL)
        def _(j):
            t = su_v[pl.ds(j, VL)] + si_v[pl.ds(j, VL)]
            o_v[pl.ds(j, VL)] = 1.0 / (1.0 + jnp.exp(-t))

        pltpu.sync_copy(o_v, o_hbm.at[pl.ds(base, BPW)])

    return k(user_ids, item_ids, score_u, score_i)


def kernel(user_ids, item_ids, user_table, item_table, fc_w, fc_b):
    ut_t = user_table.T
    it_t = item_table.T
    score_u, score_i = _tc_scores(ut_t, it_t, fc_w, fc_b)
    out = _sc_gather_head(user_ids.astype(jnp.int32), item_ids.astype(jnp.int32),
                          score_u, score_i)
    return out.reshape(B, 1)


# speedup vs baseline: 8.8921x; 1.0089x over previous
import functools

import jax
import jax.numpy as jnp
from jax import lax
from jax.experimental import pallas as pl
from jax.experimental.pallas import tpu as pltpu
from jax.experimental.pallas import tpu_sc as plsc

B = 16384
D = 32
V = 1000000
NC = 2
NS = 16
NW = NC * NS
BPW = B // NW
VBLK = 32768
VL = 16


def _tc_scores_body(ut_ref, it_ref, wu_ref, wi_ref, b_ref, su_ref, si_ref):
    su_ref[...] = jnp.sum(ut_ref[...] * wu_ref[...], axis=0) + b_ref[0]
    si_ref[...] = jnp.sum(it_ref[...] * wi_ref[...], axis=0)


def _tc_scores(ut_t, it_t, fc_w, fc_b):
    wu = fc_w[0, :D].reshape(D, 1)
    wi = fc_w[0, D:].reshape(D, 1)
    grid = (V + VBLK - 1) // VBLK
    return pl.pallas_call(
        _tc_scores_body,
        grid=(grid,),
        in_specs=[
            pl.BlockSpec((D, VBLK), lambda i: (0, i)),
            pl.BlockSpec((D, VBLK), lambda i: (0, i)),
            pl.BlockSpec((D, 1), lambda i: (0, 0)),
            pl.BlockSpec((D, 1), lambda i: (0, 0)),
            pl.BlockSpec(memory_space=pltpu.SMEM),
        ],
        out_specs=(
            pl.BlockSpec((VBLK,), lambda i: (i,)),
            pl.BlockSpec((VBLK,), lambda i: (i,)),
        ),
        out_shape=(
            jax.ShapeDtypeStruct((V,), jnp.float32),
            jax.ShapeDtypeStruct((V,), jnp.float32),
        ),
    )(ut_t, it_t, wu, wi, fc_b)


def _sc_gather_head(user_ids, item_ids, score_u, score_i):
    mesh = plsc.VectorSubcoreMesh(core_axis_name="c", subcore_axis_name="s")

    @functools.partial(
        pl.kernel,
        mesh=mesh,
        out_type=jax.ShapeDtypeStruct((B,), jnp.float32),
        scratch_types=[
            pltpu.VMEM((BPW,), jnp.int32),
            pltpu.VMEM((BPW,), jnp.int32),
            pltpu.VMEM((BPW,), jnp.float32),
            pltpu.VMEM((BPW,), jnp.float32),
            pltpu.VMEM((BPW,), jnp.float32),
            pltpu.SemaphoreType.DMA,
            pltpu.SemaphoreType.DMA,
        ],
    )
    def k(uid_hbm, iid_hbm, su_hbm, si_hbm, o_hbm,
          uidx_v, iidx_v, su_v, si_v, o_v, sem_u, sem_i):
        wid = lax.axis_index("s") * NC + lax.axis_index("c")
        base = wid * BPW
        pltpu.sync_copy(uid_hbm.at[pl.ds(base, BPW)], uidx_v)
        pltpu.sync_copy(iid_hbm.at[pl.ds(base, BPW)], iidx_v)
        cu = pltpu.async_copy(su_hbm.at[uidx_v], su_v, sem_u)
        ci = pltpu.async_copy(si_hbm.at[iidx_v], si_v, sem_i)
        cu.wait()
        ci.wait()

        @pl.loop(0, BPW, step=VL)
        def _(j):
            t = su_v[pl.ds(j, VL)] + si_v[pl.ds(j, VL)]
            o_v[pl.ds(j, VL)] = 1.0 / (1.0 + jnp.exp(-t))

        pltpu.sync_copy(o_v, o_hbm.at[pl.ds(base, BPW)])

    return k(user_ids, item_ids, score_u, score_i)


def kernel(user_ids, item_ids, user_table, item_table, fc_w, fc_b):
    ut_t = user_table.T
    it_t = item_table.T
    score_u, score_i = _tc_scores(ut_t, it_t, fc_w, fc_b)
    out = _sc_gather_head(user_ids.astype(jnp.int32), item_ids.astype(jnp.int32),
                          score_u, score_i)
    return out.reshape(B, 1)
